# Initial kernel scaffold; baseline (speedup 1.0000x reference)
import jax
import jax.numpy as jnp
from jax.experimental import pallas as pl


def kernel(inputs, edge_index, W1, b1, W2, b2):
    raise NotImplementedError("write your pallas kernel here")


# speedup vs baseline: 3.0570x; 3.0570x over previous
import jax
import jax.numpy as jnp
from jax import lax
from jax.experimental import pallas as pl
from jax.experimental.pallas import tpu as pltpu
from jax.experimental.pallas import tpu_sc as plsc

N_NODES = 10000
N_EDGES = 160000
D = 256
H = 128

NC = 2
NS = 16
K = 128

NP = 10240
RPT = NP // NS
EPT = 10240
CH = EPT // K
EP = EPT * NS
TRASH = N_NODES

RB = 1280
GRID = NP // RB


def _sc_mesh():
    return plsc.VectorSubcoreMesh(core_axis_name="c", subcore_axis_name="s")


def _deg_body(s3, d3, oz, deg, acc, idx_v, oz_v):
    c = lax.axis_index("c")
    s = lax.axis_index("s")

    @pl.when(c == 0)
    def _():
        pltpu.sync_copy(s3.at[s], idx_v)

    @pl.when(c == 1)
    def _():
        pltpu.sync_copy(d3.at[s], idx_v)

    pltpu.sync_copy(oz, oz_v)
    zz = oz_v.at[pl.ds(K, K)]
    for k in range(RPT // K):
        pltpu.sync_copy(zz, acc.at[pl.ds(s * RPT + k * K, K)])
    plsc.subcore_barrier()

    ones_rows = oz_v.at[pl.ds(0, K)]

    def body(j, carry):
        pltpu.sync_copy(ones_rows, acc.at[idx_v.at[j]], add=True)
        return carry

    lax.fori_loop(0, CH, body, 0)
    plsc.subcore_barrier()

    pltpu.sync_copy(acc.at[pl.ds(s * RPT, RPT)],
                    deg.at[pl.ds(c * NP + s * RPT, RPT)])


def _sc_degrees(s3, d3, oz):
    return pl.kernel(
        _deg_body,
        out_type=jax.ShapeDtypeStruct((2 * NP, H), jnp.float32),
        mesh=_sc_mesh(),
        scratch_types=[
            pltpu.VMEM_SHARED((NP, H), jnp.float32),
            pltpu.VMEM((CH, K), jnp.int32),
            pltpu.VMEM((2 * K, H), jnp.float32),
        ],
    )(s3, d3, oz)


def _agg_body(m, s3a, s3b, d3, z, agg, acc, src_v, dst_v, rows_v, sem):
    c = lax.axis_index("c")
    s = lax.axis_index("s")

    @pl.when(c == 0)
    def _():
        pltpu.sync_copy(s3a.at[s], src_v)

    @pl.when(c == 1)
    def _():
        pltpu.sync_copy(s3b.at[s], src_v)

    pltpu.sync_copy(d3.at[s], dst_v)

    pltpu.sync_copy(z, rows_v)
    for k in range(RPT // K):
        pltpu.sync_copy(rows_v, acc.at[pl.ds(s * RPT + k * K, K)])
    plsc.subcore_barrier()

    def body(j, carry):
        pltpu.async_copy(m.at[src_v.at[j]], rows_v, sem).wait()
        pltpu.sync_copy(rows_v, acc.at[dst_v.at[j]], add=True)
        return carry

    lax.fori_loop(0, CH, body, 0)
    plsc.subcore_barrier()

    pltpu.sync_copy(acc.at[pl.ds(s * RPT, RPT)],
                    agg.at[pl.ds(c * NP + s * RPT, RPT)])


def _sc_aggregate(m_flat, s3a, s3b, d3, z):
    return pl.kernel(
        _agg_body,
        out_type=jax.ShapeDtypeStruct((2 * NP, H), jnp.float32),
        mesh=_sc_mesh(),
        scratch_types=[
            pltpu.VMEM_SHARED((NP, H), jnp.float32),
            pltpu.VMEM((CH, K), jnp.int32),
            pltpu.VMEM((CH, K), jnp.int32),
            pltpu.VMEM((K, H), jnp.float32),
            pltpu.SemaphoreType.DMA,
        ],
    )(m_flat, s3a, s3b, d3, z)


def _norm(deg_blk):
    return lax.rsqrt(jnp.maximum(deg_blk[:, 0:1], 1.0))


def _tcA_body(x_ref, w_ref, deg_ref, out_ref):
    p = jnp.dot(x_ref[:, :], w_ref[:, :], preferred_element_type=jnp.float32)
    p = p * _norm(deg_ref)
    out_ref[0] = p[:, :H]
    out_ref[1] = p[:, H:]


def _tc_pre1(x_pad, W1, deg):
    return pl.pallas_call(
        _tcA_body,
        grid=(GRID,),
        in_specs=[
            pl.BlockSpec((RB, D), lambda i: (i, 0)),
            pl.BlockSpec((D, D), lambda i: (0, 0)),
            pl.BlockSpec((RB, H), lambda i: (i, 0)),
        ],
        out_specs=pl.BlockSpec((2, RB, H), lambda i: (0, i, 0)),
        out_shape=jax.ShapeDtypeStruct((2, NP, H), jnp.float32),
    )(x_pad, W1, deg)


def _tcB_body(aa_ref, ab_ref, di_ref, do_ref, b1_ref, w_ref, out_ref):
    agg = jnp.concatenate([aa_ref[:, :], ab_ref[:, :]], axis=1)
    h = agg * _norm(di_ref) + b1_ref[:, :]
    h = jnp.maximum(h, 0.0)
    p = jnp.dot(h * _norm(do_ref), w_ref[:, :],
                preferred_element_type=jnp.float32)
    out_ref[0] = p[:, :H]
    out_ref[1] = p[:, H:]


def _tc_mid(agg1, deg, b1, W2):
    return pl.pallas_call(
        _tcB_body,
        grid=(GRID,),
        in_specs=[
            pl.BlockSpec((RB, H), lambda i: (i, 0)),
            pl.BlockSpec((RB, H), lambda i: (i + GRID, 0)),
            pl.BlockSpec((RB, H), lambda i: (i + GRID, 0)),
            pl.BlockSpec((RB, H), lambda i: (i, 0)),
            pl.BlockSpec((1, D), lambda i: (0, 0)),
            pl.BlockSpec((D, D), lambda i: (0, 0)),
        ],
        out_specs=pl.BlockSpec((2, RB, H), lambda i: (0, i, 0)),
        out_shape=jax.ShapeDtypeStruct((2, NP, H), jnp.float32),
    )(agg1, agg1, deg, deg, b1, W2)


def _tcC_body(aa_ref, ab_ref, di_ref, b2_ref, out_ref):
    agg = jnp.concatenate([aa_ref[:, :], ab_ref[:, :]], axis=1)
    out_ref[:, :] = agg * _norm(di_ref) + b2_ref[:, :]


def _tc_post(agg2, deg, b2):
    return pl.pallas_call(
        _tcC_body,
        grid=(GRID,),
        in_specs=[
            pl.BlockSpec((RB, H), lambda i: (i, 0)),
            pl.BlockSpec((RB, H), lambda i: (i + GRID, 0)),
            pl.BlockSpec((RB, H), lambda i: (i + GRID, 0)),
            pl.BlockSpec((1, D), lambda i: (0, 0)),
        ],
        out_specs=pl.BlockSpec((RB, D), lambda i: (i, 0)),
        out_shape=jax.ShapeDtypeStruct((NP, D), jnp.float32),
    )(agg2, agg2, deg, b2)


@jax.jit
def _run(inputs, edge_index, W1, b1, W2, b2):
    src = edge_index[0]
    dst = edge_index[1]

    pad = EP - N_EDGES
    sp = jnp.concatenate([src, jnp.full((pad,), TRASH, jnp.int32)])
    dp = jnp.concatenate([dst, jnp.full((pad,), TRASH, jnp.int32)])
    s3a = sp.reshape(NS, CH, K)
    s3b = (sp + NP).reshape(NS, CH, K)
    d3 = dp.reshape(NS, CH, K)

    x_pad = jnp.concatenate(
        [inputs, jnp.zeros((NP - N_NODES, D), jnp.float32)], axis=0)
    b1r = b1.reshape(1, D)
    b2r = b2.reshape(1, D)
    oz = jnp.concatenate(
        [jnp.ones((K, H), jnp.float32), jnp.zeros((K, H), jnp.float32)])
    z = jnp.zeros((K, H), jnp.float32)

    deg = _sc_degrees(s3a, d3, oz)
    m1 = _tc_pre1(x_pad, W1, deg).reshape(2 * NP, H)
    agg1 = _sc_aggregate(m1, s3a, s3b, d3, z)
    m2 = _tc_mid(agg1, deg, b1r, W2).reshape(2 * NP, H)
    agg2 = _sc_aggregate(m2, s3a, s3b, d3, z)
    out = _tc_post(agg2, deg, b2r)
    return out[:N_NODES]


def kernel(inputs, edge_index, W1, b1, W2, b2):
    return _run(inputs, edge_index, W1, b1, W2, b2)


# speedup vs baseline: 3.5200x; 1.1515x over previous
import jax
import jax.numpy as jnp
from jax import lax
from jax.experimental import pallas as pl
from jax.experimental.pallas import tpu as pltpu
from jax.experimental.pallas import tpu_sc as plsc

N_NODES = 10000
N_EDGES = 160000
D = 256
H = 128

NC = 2
NS = 16
K = 128

NP = 10240
RPT = NP // NS
EPT = 10240
CH = EPT // K
EP = EPT * NS
TRASH = N_NODES

GB = 16

RB = 1280
GRID = NP // RB


def _sc_mesh():
    return plsc.VectorSubcoreMesh(core_axis_name="c", subcore_axis_name="s")


FD = 8


def _deg_body(s3, d3, oz, deg, acc, idx_v, oz_v, sem):
    c = lax.axis_index("c")
    s = lax.axis_index("s")

    @pl.when(c == 0)
    def _():
        pltpu.sync_copy(s3.at[s], idx_v)

    @pl.when(c == 1)
    def _():
        pltpu.sync_copy(d3.at[s], idx_v)

    pltpu.sync_copy(oz, oz_v)
    zz = oz_v.at[pl.ds(K, K)]
    for k in range(RPT // K):
        pltpu.sync_copy(zz, acc.at[pl.ds(s * RPT + k * K, K)])
    plsc.subcore_barrier()

    ones_rows = oz_v.at[pl.ds(0, K)]

    @pl.loop(0, CH, step=FD)
    def _(j0):
        descs = [
            pltpu.async_copy(ones_rows, acc.at[idx_v.at[j0 + b]], sem,
                             add=True)
            for b in range(FD)
        ]
        for d in descs:
            d.wait()

    plsc.subcore_barrier()

    pltpu.sync_copy(acc.at[pl.ds(s * RPT, RPT)],
                    deg.at[pl.ds(c * NP + s * RPT, RPT)])


def _sc_degrees(s3, d3, oz):
    return pl.kernel(
        _deg_body,
        out_type=jax.ShapeDtypeStruct((2 * NP, H), jnp.float32),
        mesh=_sc_mesh(),
        scratch_types=[
            pltpu.VMEM_SHARED((NP, H), jnp.float32),
            pltpu.VMEM((CH, K), jnp.int32),
            pltpu.VMEM((2 * K, H), jnp.float32),
            pltpu.SemaphoreType.DMA,
        ],
    )(s3, d3, oz)


def _agg_body(m, s3a, s3b, d3, z, agg, acc, src_v, dst_v, b0, b1, gsem, ssem):
    c = lax.axis_index("c")
    s = lax.axis_index("s")

    pltpu.sync_copy(z, b0)
    for k in range(RPT // K):
        pltpu.sync_copy(b0, acc.at[pl.ds(s * RPT + k * K, K)])
    plsc.subcore_barrier()

    bufs = (b0, b1)

    @pl.loop(0, CH, step=GB)
    def _(j0):
        @pl.when(c == 0)
        def _():
            pltpu.sync_copy(s3a.at[s, pl.ds(j0, GB)], src_v)

        @pl.when(c == 1)
        def _():
            pltpu.sync_copy(s3b.at[s, pl.ds(j0, GB)], src_v)

        pltpu.sync_copy(d3.at[s, pl.ds(j0, GB)], dst_v)

        g = {
            t: pltpu.async_copy(m.at[src_v.at[t]], bufs[t % 2], gsem)
            for t in range(2)
        }
        sd = {}
        for t in range(GB):
            g[t].wait()
            sd[t] = pltpu.async_copy(bufs[t % 2], acc.at[dst_v.at[t]], ssem,
                                     add=True)
            if t + 2 < GB:
                sd[t].wait()
                g[t + 2] = pltpu.async_copy(m.at[src_v.at[t + 2]],
                                            bufs[t % 2], gsem)
        sd[GB - 2].wait()
        sd[GB - 1].wait()

    plsc.subcore_barrier()

    pltpu.sync_copy(acc.at[pl.ds(s * RPT, RPT)],
                    agg.at[pl.ds(c * NP + s * RPT, RPT)])


def _sc_aggregate(m_flat, s3a, s3b, d3, z):
    return pl.kernel(
        _agg_body,
        out_type=jax.ShapeDtypeStruct((2 * NP, H), jnp.float32),
        mesh=_sc_mesh(),
        scratch_types=[
            pltpu.VMEM_SHARED((NP, H), jnp.float32),
            pltpu.VMEM((GB, K), jnp.int32),
            pltpu.VMEM((GB, K), jnp.int32),
            pltpu.VMEM((K, H), jnp.float32),
            pltpu.VMEM((K, H), jnp.float32),
            pltpu.SemaphoreType.DMA,
            pltpu.SemaphoreType.DMA,
        ],
    )(m_flat, s3a, s3b, d3, z)


def _norm(deg_blk):
    return lax.rsqrt(jnp.maximum(deg_blk[:, 0:1], 1.0))


def _tcA_body(x_ref, w_ref, deg_ref, out_ref):
    p = jnp.dot(x_ref[:, :], w_ref[:, :], preferred_element_type=jnp.float32)
    p = p * _norm(deg_ref)
    out_ref[0] = p[:, :H]
    out_ref[1] = p[:, H:]


def _tc_pre1(x_pad, W1, deg):
    return pl.pallas_call(
        _tcA_body,
        grid=(GRID,),
        in_specs=[
            pl.BlockSpec((RB, D), lambda i: (i, 0)),
            pl.BlockSpec((D, D), lambda i: (0, 0)),
            pl.BlockSpec((RB, H), lambda i: (i, 0)),
        ],
        out_specs=pl.BlockSpec((2, RB, H), lambda i: (0, i, 0)),
        out_shape=jax.ShapeDtypeStruct((2, NP, H), jnp.float32),
    )(x_pad, W1, deg)


def _tcB_body(aa_ref, ab_ref, di_ref, do_ref, b1_ref, w_ref, out_ref):
    agg = jnp.concatenate([aa_ref[:, :], ab_ref[:, :]], axis=1)
    h = agg * _norm(di_ref) + b1_ref[:, :]
    h = jnp.maximum(h, 0.0)
    p = jnp.dot(h * _norm(do_ref), w_ref[:, :],
                preferred_element_type=jnp.float32)
    out_ref[0] = p[:, :H]
    out_ref[1] = p[:, H:]


def _tc_mid(agg1, deg, b1, W2):
    return pl.pallas_call(
        _tcB_body,
        grid=(GRID,),
        in_specs=[
            pl.BlockSpec((RB, H), lambda i: (i, 0)),
            pl.BlockSpec((RB, H), lambda i: (i + GRID, 0)),
            pl.BlockSpec((RB, H), lambda i: (i + GRID, 0)),
            pl.BlockSpec((RB, H), lambda i: (i, 0)),
            pl.BlockSpec((1, D), lambda i: (0, 0)),
            pl.BlockSpec((D, D), lambda i: (0, 0)),
        ],
        out_specs=pl.BlockSpec((2, RB, H), lambda i: (0, i, 0)),
        out_shape=jax.ShapeDtypeStruct((2, NP, H), jnp.float32),
    )(agg1, agg1, deg, deg, b1, W2)


def _tcC_body(aa_ref, ab_ref, di_ref, b2_ref, out_ref):
    agg = jnp.concatenate([aa_ref[:, :], ab_ref[:, :]], axis=1)
    out_ref[:, :] = agg * _norm(di_ref) + b2_ref[:, :]


def _tc_post(agg2, deg, b2):
    return pl.pallas_call(
        _tcC_body,
        grid=(GRID,),
        in_specs=[
            pl.BlockSpec((RB, H), lambda i: (i, 0)),
            pl.BlockSpec((RB, H), lambda i: (i + GRID, 0)),
            pl.BlockSpec((RB, H), lambda i: (i + GRID, 0)),
            pl.BlockSpec((1, D), lambda i: (0, 0)),
        ],
        out_specs=pl.BlockSpec((RB, D), lambda i: (i, 0)),
        out_shape=jax.ShapeDtypeStruct((NP, D), jnp.float32),
    )(agg2, agg2, deg, b2)


@jax.jit
def _run(inputs, edge_index, W1, b1, W2, b2):
    src = edge_index[0]
    dst = edge_index[1]

    pad = EP - N_EDGES
    sp = jnp.concatenate([src, jnp.full((pad,), TRASH, jnp.int32)])
    dp = jnp.concatenate([dst, jnp.full((pad,), TRASH, jnp.int32)])
    s3d = sp.reshape(NS, CH, K)
    d3d = dp.reshape(NS, CH, K)
    s3a = s3d
    s3b = (sp + NP).reshape(NS, CH, K)
    d3 = d3d

    x_pad = jnp.concatenate(
        [inputs, jnp.zeros((NP - N_NODES, D), jnp.float32)], axis=0)
    b1r = b1.reshape(1, D)
    b2r = b2.reshape(1, D)
    oz = jnp.concatenate(
        [jnp.ones((K, H), jnp.float32), jnp.zeros((K, H), jnp.float32)])
    z = jnp.zeros((K, H), jnp.float32)

    deg = _sc_degrees(s3d, d3d, oz)
    m1 = _tc_pre1(x_pad, W1, deg).reshape(2 * NP, H)
    agg1 = _sc_aggregate(m1, s3a, s3b, d3, z)
    m2 = _tc_mid(agg1, deg, b1r, W2).reshape(2 * NP, H)
    agg2 = _sc_aggregate(m2, s3a, s3b, d3, z)
    out = _tc_post(agg2, deg, b2r)
    return out[:N_NODES]


def kernel(inputs, edge_index, W1, b1, W2, b2):
    return _run(inputs, edge_index, W1, b1, W2, b2)
